# baseline (speedup 1.0000x reference)
import jax
import jax.numpy as jnp
from jax import lax
from jax.experimental import pallas as pl
from jax.experimental.pallas import tpu as pltpu

SIZE = 100000
INPUT_SIZE = 128
CH = 12800
CHUNKS = [(k * CH, CH) for k in range(7)] + [(7 * CH, SIZE - 7 * CH)]


def _body(inp_ref, conn_ref, out_ref, win_ref,
          b0, b1, o0, o1, o2, best_ref, si0, si1, so0, so1, so2):
    bufs = (b0, b1)
    isems = (si0, si1)
    LAST = len(CHUNKS) - 1
    obufs = [o0, o1] * 4
    osems = [so0, so1] * 4
    obufs[LAST] = o2
    osems[LAST] = so2
    inp = inp_ref[...].astype(jnp.float32)
    best_ref[0] = jnp.int32(-2**31 + 1)

    def start(k):
        row0, n = CHUNKS[k]
        pltpu.async_copy(conn_ref.at[pl.ds(row0, n)],
                         bufs[k % 2].at[pl.ds(0, n)], isems[k % 2])

    start(0)
    start(1)

    for k, (row0, n) in enumerate(CHUNKS):
        pltpu.make_async_copy(conn_ref.at[pl.ds(row0, n)],
                              bufs[k % 2].at[pl.ds(0, n)], isems[k % 2]).wait()
        ov = lax.dot_general(inp, bufs[k % 2][...], (((1,), (1,)), ((), ())),
                             preferred_element_type=jnp.float32)

        flat = row0 + lax.broadcasted_iota(jnp.int32, (1, CH), 1)
        key = (ov.astype(jnp.int32) << 17) | (131071 - flat)
        if n < CH:
            key = jnp.where(flat < SIZE, key, jnp.int32(-2**31 + 1))
        best_ref[0] = jnp.maximum(best_ref[0], jnp.max(key))

        if 2 <= k <= LAST - 1:
            pr, pn = CHUNKS[k - 2]
            pltpu.make_async_copy(obufs[k - 2],
                                  out_ref.at[:, pl.ds(pr, pn)],
                                  osems[k - 2]).wait()
        obufs[k][...] = ov[:, :n]
        pltpu.async_copy(obufs[k], out_ref.at[:, pl.ds(row0, n)], osems[k])
        if k + 2 < len(CHUNKS):
            start(k + 2)

    for k in (LAST - 2, LAST - 1, LAST):
        row0, n = CHUNKS[k]
        pltpu.make_async_copy(obufs[k], out_ref.at[:, pl.ds(row0, n)],
                              osems[k]).wait()

    win_ref[0] = 131071 - (best_ref[0] & 131071)


def kernel(input_array, connected):
    inp = input_array.reshape(1, INPUT_SIZE)
    ov2d, winner1 = pl.pallas_call(
        _body,
        in_specs=[
            pl.BlockSpec((1, INPUT_SIZE), lambda: (0, 0)),
            pl.BlockSpec(memory_space=pltpu.HBM),
        ],
        out_specs=[
            pl.BlockSpec(memory_space=pltpu.HBM),
            pl.BlockSpec(memory_space=pltpu.SMEM),
        ],
        out_shape=[
            jax.ShapeDtypeStruct((1, SIZE), jnp.float32),
            jax.ShapeDtypeStruct((1,), jnp.int32),
        ],
        scratch_shapes=[
            pltpu.VMEM((CH, INPUT_SIZE), jnp.float32),
            pltpu.VMEM((CH, INPUT_SIZE), jnp.float32),
            pltpu.VMEM((1, CH), jnp.float32),
            pltpu.VMEM((1, CH), jnp.float32),
            pltpu.VMEM((1, SIZE - 7 * CH), jnp.float32),
            pltpu.SMEM((1,), jnp.int32),
            pltpu.SemaphoreType.DMA,
            pltpu.SemaphoreType.DMA,
            pltpu.SemaphoreType.DMA,
            pltpu.SemaphoreType.DMA,
            pltpu.SemaphoreType.DMA,
        ],
    )(inp, connected)
    return ov2d.reshape(SIZE), winner1[0]
